# baseline (speedup 1.0000x reference)
import jax
import jax.numpy as jnp
from jax.experimental import pallas as pl
from jax.experimental.pallas import tpu as pltpu

_T, _H, _E, _FF = 2048, 2048, 8, 704
_EPS = 1e-6
_RBT = 512


def _router_kernel(x_ref, rs_ref, rp_ref, xb_ref, comb_ref):
    x = x_ref[...]
    xb_ref[...] = x.astype(jnp.bfloat16)
    nrm = x * jax.lax.rsqrt(jnp.mean(x * x, axis=-1, keepdims=True) + _EPS)
    nrm = nrm * jnp.float32(_H ** -0.5)
    nrm = nrm * rs_ref[...]
    scores = jax.lax.dot_general(
        nrm, rp_ref[...], (((1,), (1,)), ((), ())),
        preferred_element_type=jnp.float32)
    m = jnp.max(scores, axis=-1, keepdims=True)
    p = jnp.exp(scores - m)
    lane = jax.lax.broadcasted_iota(jnp.int32, p.shape, 1)
    p0 = jnp.max(p, axis=-1, keepdims=True)
    i0 = jnp.min(jnp.where(p == p0, lane, _E), axis=-1, keepdims=True)
    pm = jnp.where(lane == i0, -1.0, p)
    p1 = jnp.max(pm, axis=-1, keepdims=True)
    i1 = jnp.min(jnp.where(pm == p1, lane, _E), axis=-1, keepdims=True)
    den = p0 + p1
    w0 = p0 / den
    w1v = p1 / den
    comb_ref[...] = w0 * (lane == i0) + w1v * (lane == i1)


def _moe_fused_kernel(xb_ref, comb_ref, w1_ref, w3_ref, w2_ref, out_ref):
    e = pl.program_id(0)
    f = pl.program_id(1)
    w1b = w1_ref[0].astype(jnp.bfloat16)
    w3b = w3_ref[0].astype(jnp.bfloat16)
    lane = jax.lax.broadcasted_iota(jnp.int32, (_T // 4, _E), 1)
    for tc in range(4):
        sl = pl.ds(tc * (_T // 4), _T // 4)
        xb = xb_ref[sl, :]
        comb = comb_ref[sl, :]
        ce = jnp.sum(jnp.where(lane == e, comb, 0.0), axis=-1, keepdims=True)
        g = jax.lax.dot_general(xb, w1b, (((1,), (1,)), ((), ())),
                                preferred_element_type=jnp.float32)
        u = jax.lax.dot_general(xb, w3b, (((1,), (1,)), ((), ())),
                                preferred_element_type=jnp.float32)
        h = (jax.nn.gelu(g, approximate=True) * u * ce).astype(jnp.bfloat16)

        @pl.when(f == 0)
        def _f0():
            w2h = w2_ref[0, :, 0:_FF // 2].astype(jnp.bfloat16)
            y = jax.lax.dot_general(h, w2h, (((1,), (1,)), ((), ())),
                                    preferred_element_type=jnp.float32)

            @pl.when(e == 0)
            def _init():
                out_ref[sl, :] = y

            @pl.when(e != 0)
            def _acc():
                out_ref[sl, :] += y

        @pl.when(f == 1)
        def _f1():
            w2h = w2_ref[0, :, _FF // 2:_FF].astype(jnp.bfloat16)
            y = jax.lax.dot_general(h, w2h, (((1,), (1,)), ((), ())),
                                    preferred_element_type=jnp.float32)
            out_ref[sl, :] += y


def kernel(hidden_states, router_scale, router_proj, w1, w2, w3):
    rs2 = router_scale.reshape(1, _H)
    xb, comb = pl.pallas_call(
        _router_kernel,
        grid=(_T // _RBT,),
        in_specs=[
            pl.BlockSpec((_RBT, _H), lambda t: (t, 0)),
            pl.BlockSpec((1, _H), lambda t: (0, 0)),
            pl.BlockSpec((_E, _H), lambda t: (0, 0)),
        ],
        out_specs=[
            pl.BlockSpec((_RBT, _H), lambda t: (t, 0)),
            pl.BlockSpec((_RBT, _E), lambda t: (t, 0)),
        ],
        out_shape=[
            jax.ShapeDtypeStruct((_T, _H), jnp.bfloat16),
            jax.ShapeDtypeStruct((_T, _E), jnp.float32),
        ],
    )(hidden_states, rs2, router_proj)

    out = pl.pallas_call(
        _moe_fused_kernel,
        grid=(_E, 2),
        in_specs=[
            pl.BlockSpec((_T, _H), lambda e, f: (0, 0)),
            pl.BlockSpec((_T, _E), lambda e, f: (0, 0)),
            pl.BlockSpec((1, _FF // 2, _H), lambda e, f: (e, f, 0)),
            pl.BlockSpec((1, _FF // 2, _H), lambda e, f: (e, f, 0)),
            pl.BlockSpec((1, _H, _FF), lambda e, f: (e, 0, 0)),
        ],
        out_specs=pl.BlockSpec((_T, _H), lambda e, f: (0, 0)),
        out_shape=jax.ShapeDtypeStruct((_T, _H), jnp.float32),
        compiler_params=pltpu.CompilerParams(
            dimension_semantics=("arbitrary", "arbitrary"),
            vmem_limit_bytes=100 * 1024 * 1024,
        ),
    )(xb, comb, w1, w3, w2)
    return out


# speedup vs baseline: 1.2409x; 1.2409x over previous
import jax
import jax.numpy as jnp
from jax.experimental import pallas as pl
from jax.experimental.pallas import tpu as pltpu

_T, _H, _E, _FF = 2048, 2048, 8, 704
_EPS = 1e-6
_RBT = 512


def _router_kernel(x_ref, rs_ref, rp_ref, xb_ref, i0_ref, i1_ref, tw0_ref, tw1_ref):
    x = x_ref[...]
    xb_ref[...] = x.astype(jnp.bfloat16)
    nrm = x * jax.lax.rsqrt(jnp.mean(x * x, axis=-1, keepdims=True) + _EPS)
    nrm = nrm * jnp.float32(_H ** -0.5)
    nrm = nrm * rs_ref[...]
    scores = jax.lax.dot_general(
        nrm, rp_ref[...], (((1,), (1,)), ((), ())),
        preferred_element_type=jnp.float32)
    m = jnp.max(scores, axis=-1, keepdims=True)
    p = jnp.exp(scores - m)
    lane = jax.lax.broadcasted_iota(jnp.int32, p.shape, 1)
    p0 = jnp.max(p, axis=-1, keepdims=True)
    i0 = jnp.min(jnp.where(p == p0, lane, _E), axis=-1, keepdims=True)
    pm = jnp.where(lane == i0, -1.0, p)
    p1 = jnp.max(pm, axis=-1, keepdims=True)
    i1 = jnp.min(jnp.where(pm == p1, lane, _E), axis=-1, keepdims=True)
    den = p0 + p1
    i0_ref[...] = i0
    i1_ref[...] = i1
    tw0_ref[...] = p0 / den
    tw1_ref[...] = p1 / den


def _gateup_kernel(xb_ref, i0_ref, i1_ref, tw0_ref, tw1_ref,
                   w1_ref, w3_ref, h_ref):
    e = pl.program_id(0)
    ce = jnp.where(i0_ref[...] == e, tw0_ref[...],
                   jnp.where(i1_ref[...] == e, tw1_ref[...], 0.0))
    xb = xb_ref[...]
    w1b = w1_ref[0].astype(jnp.bfloat16)
    w3b = w3_ref[0].astype(jnp.bfloat16)
    g = jax.lax.dot_general(xb, w1b, (((1,), (1,)), ((), ())),
                            preferred_element_type=jnp.float32)
    u = jax.lax.dot_general(xb, w3b, (((1,), (1,)), ((), ())),
                            preferred_element_type=jnp.float32)
    h = jax.nn.gelu(g, approximate=True) * u * ce
    h_ref[0] = h.astype(jnp.bfloat16)


def _down_kernel(h_ref, w2_ref, out_ref):
    e = pl.program_id(0)
    w2b = w2_ref[0].astype(jnp.bfloat16)
    y = jax.lax.dot_general(h_ref[0], w2b, (((1,), (1,)), ((), ())),
                            preferred_element_type=jnp.float32)

    @pl.when(e == 0)
    def _init():
        out_ref[...] = y

    @pl.when(e != 0)
    def _acc():
        out_ref[...] += y


def kernel(hidden_states, router_scale, router_proj, w1, w2, w3):
    rs2 = router_scale.reshape(1, _H)
    xb, i0, i1, tw0, tw1 = pl.pallas_call(
        _router_kernel,
        grid=(_T // _RBT,),
        in_specs=[
            pl.BlockSpec((_RBT, _H), lambda t: (t, 0)),
            pl.BlockSpec((1, _H), lambda t: (0, 0)),
            pl.BlockSpec((_E, _H), lambda t: (0, 0)),
        ],
        out_specs=[
            pl.BlockSpec((_RBT, _H), lambda t: (t, 0)),
            pl.BlockSpec((_RBT, 1), lambda t: (t, 0)),
            pl.BlockSpec((_RBT, 1), lambda t: (t, 0)),
            pl.BlockSpec((_RBT, 1), lambda t: (t, 0)),
            pl.BlockSpec((_RBT, 1), lambda t: (t, 0)),
        ],
        out_shape=[
            jax.ShapeDtypeStruct((_T, _H), jnp.bfloat16),
            jax.ShapeDtypeStruct((_T, 1), jnp.int32),
            jax.ShapeDtypeStruct((_T, 1), jnp.int32),
            jax.ShapeDtypeStruct((_T, 1), jnp.float32),
            jax.ShapeDtypeStruct((_T, 1), jnp.float32),
        ],
    )(hidden_states, rs2, router_proj)

    hbuf = pl.pallas_call(
        _gateup_kernel,
        grid=(_E,),
        in_specs=[
            pl.BlockSpec((_T, _H), lambda e: (0, 0)),
            pl.BlockSpec((_T, 1), lambda e: (0, 0)),
            pl.BlockSpec((_T, 1), lambda e: (0, 0)),
            pl.BlockSpec((_T, 1), lambda e: (0, 0)),
            pl.BlockSpec((_T, 1), lambda e: (0, 0)),
            pl.BlockSpec((1, _FF, _H), lambda e: (e, 0, 0)),
            pl.BlockSpec((1, _FF, _H), lambda e: (e, 0, 0)),
        ],
        out_specs=pl.BlockSpec((1, _T, _FF), lambda e: (e, 0, 0)),
        out_shape=jax.ShapeDtypeStruct((_E, _T, _FF), jnp.bfloat16),
        compiler_params=pltpu.CompilerParams(
            dimension_semantics=("arbitrary",),
        ),
    )(xb, i0, i1, tw0, tw1, w1, w3)

    out = pl.pallas_call(
        _down_kernel,
        grid=(_E,),
        in_specs=[
            pl.BlockSpec((1, _T, _FF), lambda e: (e, 0, 0)),
            pl.BlockSpec((1, _H, _FF), lambda e: (e, 0, 0)),
        ],
        out_specs=pl.BlockSpec((_T, _H), lambda e: (0, 0)),
        out_shape=jax.ShapeDtypeStruct((_T, _H), jnp.float32),
        compiler_params=pltpu.CompilerParams(
            dimension_semantics=("arbitrary",),
        ),
    )(hbuf, w2)
    return out
